# Initial kernel scaffold; baseline (speedup 1.0000x reference)
import jax
import jax.numpy as jnp
from jax.experimental import pallas as pl


def kernel(src, tok_table, pos_table):
    raise NotImplementedError("write your pallas kernel here")


# speedup vs baseline: 2.6163x; 2.6163x over previous
import functools
import jax
import jax.numpy as jnp
from jax import lax
from jax.experimental import pallas as pl
from jax.experimental.pallas import tpu as pltpu
from jax.experimental.pallas import tpu_sc as plsc

HID = 128
LANES = 16
NC = 2
NS = 16
NW = NC * NS


def _make_kernel(B, S):
    rows = B * S
    chunk = 4 * S
    gat = 2 * S
    assert rows % (NW * chunk) == 0
    rows_per_w = rows // NW
    nchunks = rows_per_w // chunk
    idx_rows_per_w = rows_per_w // gat

    mesh = plsc.VectorSubcoreMesh(core_axis_name="c", subcore_axis_name="s")

    @functools.partial(
        pl.kernel,
        mesh=mesh,
        out_type=jax.ShapeDtypeStruct((rows, HID), jnp.float32),
        scratch_types=[
            pltpu.VMEM((idx_rows_per_w, gat), jnp.int32),
            pltpu.VMEM((S, HID), jnp.float32),
            pltpu.VMEM((chunk, HID), jnp.float32),
            pltpu.SemaphoreType.DMA,
        ],
    )
    def k(src_hbm, tok_hbm, pos_hbm, out_hbm, idx_v, pos_v, buf_v, sem):
        c = lax.axis_index("c")
        s = lax.axis_index("s")
        wid = s * NC + c
        base = wid * rows_per_w

        pltpu.sync_copy(src_hbm.at[pl.ds(wid * idx_rows_per_w, idx_rows_per_w)],
                        idx_v)
        pltpu.sync_copy(pos_hbm, pos_v)

        scale = jnp.float32(HID ** 0.5)

        def chunk_body(g, carry):
            cp0 = pltpu.async_copy(tok_hbm.at[idx_v.at[2 * g]],
                                   buf_v.at[pl.ds(0, gat)], sem)
            cp1 = pltpu.async_copy(tok_hbm.at[idx_v.at[2 * g + 1]],
                                   buf_v.at[pl.ds(gat, gat)], sem)
            cp0.wait()
            cp1.wait()

            def seq_body(sq, carry2):
                for q in range(chunk // S):
                    r = q * S + sq
                    for j in range(HID // LANES):
                        col = pl.ds(j * LANES, LANES)
                        buf_v[r, col] = buf_v[r, col] * scale + pos_v[sq, col]
                return carry2

            lax.fori_loop(0, S, seq_body, 0)
            pltpu.sync_copy(buf_v, out_hbm.at[pl.ds(base + g * chunk, chunk)])
            return carry

        lax.fori_loop(0, nchunks, chunk_body, 0)

    return k


def kernel(src, tok_table, pos_table):
    B, S = src.shape
    idx = jnp.asarray(src, jnp.int32).reshape(-1, 2 * S)
    out = _make_kernel(B, S)(idx, tok_table, pos_table[:S])
    return out.reshape(B, S, HID)


# speedup vs baseline: 3.2641x; 1.2476x over previous
import functools
import jax
import jax.numpy as jnp
from jax import lax
from jax.experimental import pallas as pl
from jax.experimental.pallas import tpu as pltpu
from jax.experimental.pallas import tpu_sc as plsc

HID = 128
LANES = 16
NC = 2
NS = 16
NW = NC * NS
NBUF = 4


def _make_kernel(B, S):
    rows = B * S
    chunk = 4 * S
    gat = 2 * S
    assert rows % (NW * chunk) == 0
    rows_per_w = rows // NW
    nchunks = rows_per_w // chunk
    assert nchunks % NBUF == 0
    idx_rows_per_w = rows_per_w // gat

    mesh = plsc.VectorSubcoreMesh(core_axis_name="c", subcore_axis_name="s")

    @functools.partial(
        pl.kernel,
        mesh=mesh,
        out_type=jax.ShapeDtypeStruct((rows, HID), jnp.float32),
        scratch_types=[
            pltpu.VMEM((idx_rows_per_w, gat), jnp.int32),
            pltpu.VMEM((S, HID), jnp.float32),
        ] + [pltpu.VMEM((chunk, HID), jnp.float32) for _ in range(NBUF)] + [
            pltpu.SemaphoreType.DMA((NBUF,)),
            pltpu.SemaphoreType.DMA((NBUF,)),
        ],
    )
    def k(src_hbm, tok_hbm, pos_hbm, out_hbm, idx_v, pos_v, b0, b1, b2, b3,
          gsem, ssem):
        bufs = [b0, b1, b2, b3]
        c = lax.axis_index("c")
        s = lax.axis_index("s")
        wid = s * NC + c
        base = wid * rows_per_w

        pltpu.sync_copy(src_hbm.at[pl.ds(wid * idx_rows_per_w, idx_rows_per_w)],
                        idx_v)
        pltpu.sync_copy(pos_hbm, pos_v)

        scale = jnp.float32(HID ** 0.5)

        def issue_gather(g, b):
            pltpu.async_copy(tok_hbm.at[idx_v.at[2 * g]],
                             bufs[b].at[pl.ds(0, gat)], gsem.at[b])
            pltpu.async_copy(tok_hbm.at[idx_v.at[2 * g + 1]],
                             bufs[b].at[pl.ds(gat, gat)], gsem.at[b])

        def wait_gather(g, b):
            pltpu.make_async_copy(tok_hbm.at[idx_v.at[2 * g]],
                                  bufs[b].at[pl.ds(0, gat)], gsem.at[b]).wait()
            pltpu.make_async_copy(tok_hbm.at[idx_v.at[2 * g + 1]],
                                  bufs[b].at[pl.ds(gat, gat)], gsem.at[b]).wait()

        def out_slice(g):
            return out_hbm.at[pl.ds(base + g * chunk, chunk)]

        def wait_store(g, b):
            pltpu.make_async_copy(bufs[b], out_slice(g), ssem.at[b]).wait()

        issue_gather(jnp.int32(0), 0)
        issue_gather(jnp.int32(1), 1)

        def outer_body(i, carry):
            for b in range(NBUF):
                g = i * NBUF + b
                nb = (b + 2) % NBUF

                @pl.when(g + 2 < nchunks)
                def _():
                    @pl.when(g >= 2)
                    def _():
                        wait_store(g - 2, nb)
                    issue_gather(g + 2, nb)

                wait_gather(g, b)

                buf = bufs[b]

                @plsc.parallel_loop(0, S, unroll=2)
                def fma(sq):
                    for q in range(chunk // S):
                        r = q * S + sq
                        for j in range(HID // LANES):
                            col = pl.ds(j * LANES, LANES)
                            buf[r, col] = buf[r, col] * scale + pos_v[sq, col]

                pltpu.async_copy(buf, out_slice(g), ssem.at[b])
            return carry

        lax.fori_loop(0, nchunks // NBUF, outer_body, 0)

        for j in range(NBUF):
            g = nchunks - NBUF + j
            wait_store(g, g % NBUF)

    return k


def kernel(src, tok_table, pos_table):
    B, S = src.shape
    idx = jnp.asarray(src, jnp.int32).reshape(-1, 2 * S)
    out = _make_kernel(B, S)(idx, tok_table, pos_table[:S])
    return out.reshape(B, S, HID)


# speedup vs baseline: 5.7433x; 1.7596x over previous
import functools
import jax
import jax.numpy as jnp
from jax import lax
from jax.experimental import pallas as pl
from jax.experimental.pallas import tpu as pltpu
from jax.experimental.pallas import tpu_sc as plsc

HID = 128
LANES = 16
NC = 2
NS = 16
NW = NC * NS
NBUF = 4
SEQ_PER_CHUNK = 2


def _make_kernel(B, S):
    assert B % NW == 0
    seqs_per_w = B // NW
    assert seqs_per_w % SEQ_PER_CHUNK == 0
    nchunks = seqs_per_w // SEQ_PER_CHUNK
    assert nchunks % NBUF == 0

    mesh = plsc.VectorSubcoreMesh(core_axis_name="c", subcore_axis_name="s")

    @functools.partial(
        pl.kernel,
        mesh=mesh,
        out_type=jax.ShapeDtypeStruct((B, S, HID), jnp.float32),
        scratch_types=[
            pltpu.VMEM((seqs_per_w, S), jnp.int32),
            pltpu.VMEM((S, HID), jnp.float32),
        ] + [pltpu.VMEM((SEQ_PER_CHUNK, S, HID), jnp.float32)
             for _ in range(NBUF)] + [
            pltpu.SemaphoreType.DMA((NBUF,)),
            pltpu.SemaphoreType.DMA((NBUF,)),
        ],
    )
    def k(src_hbm, tok_hbm, pos_hbm, out_hbm, idx_v, pos_v, b0, b1, b2, b3,
          gsem, ssem):
        bufs = [b0, b1, b2, b3]
        c = lax.axis_index("c")
        s = lax.axis_index("s")
        wid = s * NC + c
        base = wid * seqs_per_w

        pltpu.sync_copy(src_hbm.at[pl.ds(base, seqs_per_w)], idx_v)
        pltpu.sync_copy(pos_hbm, pos_v)

        scale = jnp.float32(HID ** 0.5)

        def issue_gather(g, b):
            for q in range(SEQ_PER_CHUNK):
                pltpu.async_copy(tok_hbm.at[idx_v.at[g * SEQ_PER_CHUNK + q]],
                                 bufs[b].at[q], gsem.at[b])

        def wait_gather(g, b):
            for q in range(SEQ_PER_CHUNK):
                pltpu.make_async_copy(
                    tok_hbm.at[idx_v.at[g * SEQ_PER_CHUNK + q]],
                    bufs[b].at[q], gsem.at[b]).wait()

        def out_slice(g):
            return out_hbm.at[pl.ds(base + g * SEQ_PER_CHUNK, SEQ_PER_CHUNK)]

        def wait_store(g, b):
            pltpu.make_async_copy(bufs[b], out_slice(g), ssem.at[b]).wait()

        issue_gather(jnp.int32(0), 0)
        issue_gather(jnp.int32(1), 1)

        def outer_body(i, carry):
            for b in range(NBUF):
                g = i * NBUF + b
                nb = (b + 2) % NBUF

                @pl.when(g + 2 < nchunks)
                def _():
                    @pl.when(g >= 2)
                    def _():
                        wait_store(g - 2, nb)
                    issue_gather(g + 2, nb)

                wait_gather(g, b)

                buf = bufs[b]

                @plsc.parallel_loop(0, S, unroll=2)
                def fma(sq):
                    for q in range(SEQ_PER_CHUNK):
                        for j in range(HID // LANES):
                            col = pl.ds(j * LANES, LANES)
                            buf[q, sq, col] = (buf[q, sq, col] * scale
                                               + pos_v[sq, col])

                pltpu.async_copy(buf, out_slice(g), ssem.at[b])
            return carry

        lax.fori_loop(0, nchunks // NBUF, outer_body, 0)

        for j in range(NBUF):
            g = nchunks - NBUF + j
            wait_store(g, g % NBUF)

    return k


def kernel(src, tok_table, pos_table):
    B, S = src.shape
    idx = jnp.asarray(src, jnp.int32)
    return _make_kernel(B, S)(idx, tok_table, pos_table[:S])


# speedup vs baseline: 10.4129x; 1.8130x over previous
import functools
import jax
import jax.numpy as jnp
from jax import lax
from jax.experimental import pallas as pl
from jax.experimental.pallas import tpu as pltpu
from jax.experimental.pallas import tpu_sc as plsc

HID = 128
LANES = 16
NC = 2
NS = 16
NW = NC * NS
NBUF = 5


def _make_kernel(B, S):
    assert B % NW == 0
    bs_per_w = B // NW
    assert bs_per_w % 8 == 0
    nchunks = S
    assert nchunks % NBUF == 0

    mesh = plsc.VectorSubcoreMesh(core_axis_name="c", subcore_axis_name="s")

    @functools.partial(
        pl.kernel,
        mesh=mesh,
        out_type=jax.ShapeDtypeStruct((S, B, HID), jnp.float32),
        scratch_types=[
            pltpu.VMEM((S, bs_per_w), jnp.int32),
            pltpu.VMEM((S, HID), jnp.float32),
        ] + [pltpu.VMEM((bs_per_w, HID), jnp.float32)
             for _ in range(NBUF)] + [
            pltpu.SemaphoreType.DMA((NBUF,)),
            pltpu.SemaphoreType.DMA((NBUF,)),
        ],
    )
    def k(srct_hbm, tok_hbm, pos_hbm, out_hbm, idx_v, pos_v, b0, b1, b2, b3,
          b4, gsem, ssem):
        bufs = [b0, b1, b2, b3, b4]
        c = lax.axis_index("c")
        s = lax.axis_index("s")
        wid = s * NC + c
        base = wid * bs_per_w

        pltpu.sync_copy(srct_hbm.at[:, pl.ds(base, bs_per_w)], idx_v)
        pltpu.sync_copy(pos_hbm, pos_v)

        scale = jnp.float32(HID ** 0.5)

        def issue_gather(g, b):
            pltpu.async_copy(tok_hbm.at[idx_v.at[g]], bufs[b], gsem.at[b])

        def wait_gather(g, b):
            pltpu.make_async_copy(tok_hbm.at[idx_v.at[g]], bufs[b],
                                  gsem.at[b]).wait()

        def out_slice(g):
            return out_hbm.at[g, pl.ds(base, bs_per_w)]

        def wait_store(g, b):
            pltpu.make_async_copy(bufs[b], out_slice(g), ssem.at[b]).wait()

        issue_gather(jnp.int32(0), 0)
        issue_gather(jnp.int32(1), 1)

        def outer_body(i, carry):
            for b in range(NBUF):
                g = i * NBUF + b
                nb = (b + 2) % NBUF

                @pl.when(g + 2 < nchunks)
                def _():
                    @pl.when(g >= 3)
                    def _():
                        wait_store(g - 3, nb)
                    issue_gather(g + 2, nb)

                wait_gather(g, b)

                buf = bufs[b]
                pvec = [pos_v[g, pl.ds(j * LANES, LANES)]
                        for j in range(HID // LANES)]

                @plsc.parallel_loop(0, bs_per_w, unroll=2)
                def fma(r):
                    for j in range(HID // LANES):
                        col = pl.ds(j * LANES, LANES)
                        buf[r, col] = buf[r, col] * scale + pvec[j]

                pltpu.async_copy(buf, out_slice(g), ssem.at[b])
            return carry

        lax.fori_loop(0, nchunks // NBUF, outer_body, 0)

        for j in range(NBUF):
            g = nchunks - NBUF + j
            wait_store(g, g % NBUF)

    return k


def kernel(src, tok_table, pos_table):
    B, S = src.shape
    src_t = jnp.transpose(jnp.asarray(src, jnp.int32))
    out_sb = _make_kernel(B, S)(src_t, tok_table, pos_table[:S])
    return jnp.transpose(out_sb, (1, 0, 2))
